# baseline (speedup 1.0000x reference)
import functools

import jax
import jax.numpy as jnp
from jax import lax
from jax.experimental import pallas as pl
from jax.experimental.pallas import tpu as pltpu
from jax.experimental.pallas import tpu_sc as plsc

NUM_NODES = 1000000
NODE_DIM = 64
PATH_LEN = 16384

NC = 2
NS = 16
NW = NC * NS
BPW = PATH_LEN // NW
CHUNK = 128
NCHUNK = BPW // CHUNK
L = 16
NWORD = BPW * NODE_DIM
GCHUNK = 128
NGCH = NWORD // GCHUNK


def _body(path_hbm, mask_hbm, tflat_hbm, out_hbm,
          idxf_v, maskf_v, widx_v, gdst_v, packed_v, sem):
    wid = lax.axis_index("s") * NC + lax.axis_index("c")
    base = wid * BPW

    pltpu.sync_copy(path_hbm.at[pl.ds(base, BPW)], idxf_v)
    pltpu.sync_copy(mask_hbm.at[pl.ds(base, BPW)], maskf_v)

    for k in range(BPW // L):
        sl = pl.ds(k * L, L)
        m = maskf_v[sl]
        maskf_v[sl] = jnp.minimum(jnp.maximum(m, 0), 1)

    def build_d(d, _):
        off = d * NUM_NODES
        for k in range(BPW // L):
            sl = pl.ds(k * L, L)
            widx_v[pl.ds(d * BPW + k * L, L)] = idxf_v[sl] + off
        return 0

    lax.fori_loop(0, NODE_DIM, build_d, 0)

    copies = []
    for j in range(NGCH):
        copies.append(
            pltpu.async_copy(
                tflat_hbm.at[widx_v.at[pl.ds(j * GCHUNK, GCHUNK)]],
                gdst_v.at[pl.ds(j * GCHUNK, GCHUNK)],
                sem,
            )
        )
    for c in copies:
        c.wait()

    lanes512 = lax.iota(jnp.int32, L) * BPW

    def one_row(r, _):
        mf = plsc.load_gather(maskf_v, [jnp.broadcast_to(r, (L,))]).astype(jnp.float32)
        src = lanes512 + r
        for c in range(NODE_DIM // L):
            v = plsc.load_gather(gdst_v, [src + c * L * BPW])
            packed_v[r, pl.ds(c * L, L)] = v * mf
        return 0

    lax.fori_loop(0, BPW, one_row, 0)

    pltpu.sync_copy(packed_v, out_hbm.at[pl.ds(base, BPW)])


def kernel(path, mask, table):
    tflat = jnp.swapaxes(table, 0, 1).reshape(NODE_DIM * NUM_NODES)
    mesh = plsc.VectorSubcoreMesh(core_axis_name="c", subcore_axis_name="s")
    f = functools.partial(
        pl.kernel,
        mesh=mesh,
        compiler_params=pltpu.CompilerParams(
            use_tc_tiling_on_sc=False,
            needs_layout_passes=False,
        ),
        out_type=jax.ShapeDtypeStruct((PATH_LEN, NODE_DIM), jnp.float32),
        scratch_types=[
            pltpu.VMEM((BPW,), jnp.int32),
            pltpu.VMEM((BPW,), jnp.int32),
            pltpu.VMEM((NWORD,), jnp.int32),
            pltpu.VMEM((NWORD,), jnp.float32),
            pltpu.VMEM((BPW, NODE_DIM), jnp.float32),
            pltpu.SemaphoreType.DMA,
        ],
    )(_body)
    return f(path.astype(jnp.int32), mask, tflat)


# speedup vs baseline: 7.8473x; 7.8473x over previous
import functools

import jax
import jax.numpy as jnp
from jax import lax
from jax.experimental import pallas as pl
from jax.experimental.pallas import tpu as pltpu
from jax.experimental.pallas import tpu_sc as plsc

NUM_NODES = 1000000
NODE_DIM = 64
PATH_LEN = 16384

NC = 2
NS = 16
NW = NC * NS
BPW = PATH_LEN // NW
CHUNK = 128
NCHUNK = BPW // CHUNK
L = 16


def _body(path_hbm, mask_hbm, pairs_hbm, out_hbm,
          idx_v, aux_v, rows2_v, packed_v, sem):
    wid = lax.axis_index("s") * NC + lax.axis_index("c")
    base = wid * BPW

    for j in range(NCHUNK):
        pltpu.sync_copy(path_hbm.at[pl.ds(base + j * CHUNK, CHUNK)], idx_v.at[j])
        pltpu.sync_copy(mask_hbm.at[pl.ds(base + j * CHUNK, CHUNK)],
                        aux_v.at[pl.ds(j * CHUNK, CHUNK)])

    for j in range(NCHUNK):
        for k in range(CHUNK // L):
            sl = pl.ds(k * L, L)
            fl = pl.ds(j * CHUNK + k * L, L)
            v = idx_v[j, sl]
            m = aux_v[fl]
            aux_v[fl] = jnp.minimum(jnp.maximum(m, 0), 1)
            aux_v[pl.ds(BPW + j * CHUNK + k * L, L)] = (v & 1) * NODE_DIM
            idx_v[j, sl] = v >> 1

    copies = []
    for j in range(NCHUNK):
        copies.append(
            pltpu.async_copy(
                pairs_hbm.at[idx_v.at[j]],
                rows2_v.at[pl.ds(j * CHUNK, CHUNK)],
                sem,
            )
        )
    for c in copies:
        c.wait()

    lanes = lax.iota(jnp.int32, L)

    def one_row(r, _):
        mf = plsc.load_gather(aux_v, [jnp.broadcast_to(r, (L,))]).astype(jnp.float32)
        off = plsc.load_gather(aux_v, [jnp.broadcast_to(BPW + r, (L,))])
        row_idx = jnp.broadcast_to(r, (L,))
        for c in range(NODE_DIM // L):
            v = plsc.load_gather(rows2_v, [row_idx, off + c * L + lanes])
            packed_v[pl.ds(r * NODE_DIM + c * L, L)] = v * mf
        return 0

    lax.fori_loop(0, BPW, one_row, 0)

    pltpu.sync_copy(packed_v, out_hbm.at[pl.ds(base * NODE_DIM, BPW * NODE_DIM)])


def kernel(path, mask, table):
    pairs = table.reshape(NUM_NODES // 2, 2 * NODE_DIM)
    mesh = plsc.VectorSubcoreMesh(core_axis_name="c", subcore_axis_name="s")
    f = functools.partial(
        pl.kernel,
        mesh=mesh,
        compiler_params=pltpu.CompilerParams(
            use_tc_tiling_on_sc=True,
            needs_layout_passes=False,
        ),
        out_type=jax.ShapeDtypeStruct((PATH_LEN * NODE_DIM,), jnp.float32),
        scratch_types=[
            pltpu.VMEM((NCHUNK, CHUNK), jnp.int32),
            pltpu.VMEM((2 * BPW,), jnp.int32),
            pltpu.VMEM((BPW, 2 * NODE_DIM), jnp.float32),
            pltpu.VMEM((BPW * NODE_DIM,), jnp.float32),
            pltpu.SemaphoreType.DMA,
        ],
    )(_body)
    return f(path.astype(jnp.int32), mask, pairs).reshape(PATH_LEN, NODE_DIM)


# speedup vs baseline: 8.0205x; 1.0221x over previous
import functools

import jax
import jax.numpy as jnp
from jax import lax
from jax.experimental import pallas as pl
from jax.experimental.pallas import tpu as pltpu
from jax.experimental.pallas import tpu_sc as plsc

NUM_NODES = 1000000
NODE_DIM = 64
PATH_LEN = 16384

NC = 2
NS = 16
NW = NC * NS
BPW = PATH_LEN // NW
CHUNK = 128
NCHUNK = BPW // CHUNK


def _body(path_hbm, mask_hbm, table_hbm, out_hbm,
          idx_v, mask_v, maskf_v, rows_v, sem):
    wid = lax.axis_index("s") * NC + lax.axis_index("c")
    base = wid * BPW

    for j in range(NCHUNK):
        pltpu.sync_copy(path_hbm.at[pl.ds(base + j * CHUNK, CHUNK)], idx_v.at[j])
        pltpu.sync_copy(mask_hbm.at[pl.ds(base + j * CHUNK, CHUNK)], mask_v.at[j])

    copies = []
    for j in range(NCHUNK):
        copies.append(
            pltpu.async_copy(
                table_hbm.at[idx_v.at[j]],
                rows_v.at[pl.ds(j * CHUNK, CHUNK)],
                sem,
            )
        )

    for j in range(NCHUNK):
        for k in range(CHUNK // 16):
            v = mask_v[j, pl.ds(k * 16, 16)]
            v01 = jnp.minimum(jnp.maximum(v, 0), 1)
            maskf_v[pl.ds(j * CHUNK + k * 16, 16)] = v01.astype(jnp.float32)

    for c in copies:
        c.wait()

    def mask_row(r, _):
        mf = plsc.load_gather(maskf_v, [jnp.broadcast_to(r, (16,))])
        for c in range(NODE_DIM // 16):
            sl = pl.ds(c * 16, 16)
            rows_v[r, sl] = rows_v[r, sl] * mf
        return 0

    lax.fori_loop(0, BPW, mask_row, 0)

    pltpu.sync_copy(rows_v, out_hbm.at[pl.ds(base, BPW)])


def kernel(path, mask, table):
    mesh = plsc.VectorSubcoreMesh(core_axis_name="c", subcore_axis_name="s")
    f = functools.partial(
        pl.kernel,
        mesh=mesh,
        compiler_params=pltpu.CompilerParams(
            use_tc_tiling_on_sc=False,
            needs_layout_passes=False,
        ),
        out_type=jax.ShapeDtypeStruct((PATH_LEN, NODE_DIM), jnp.float32),
        scratch_types=[
            pltpu.VMEM((NCHUNK, CHUNK), jnp.int32),
            pltpu.VMEM((NCHUNK, CHUNK), jnp.int32),
            pltpu.VMEM((BPW,), jnp.float32),
            pltpu.VMEM((BPW, NODE_DIM), jnp.float32),
            pltpu.SemaphoreType.DMA,
        ],
    )(_body)
    return f(path.astype(jnp.int32), mask, table)
